# baseline (speedup 1.0000x reference)
import functools

import jax
import jax.numpy as jnp
from jax import lax
from jax.experimental import pallas as pl
from jax.experimental.pallas import tpu as pltpu
from jax.experimental.pallas import tpu_sc as plsc

VOCAB = 1000000
EMBED = 64
BATCH = 16384
HIST = 50

NW = 32
TOTAL = BATCH * HIST
C = 128
NCHUNK_TOTAL = TOTAL // C
NCHUNK = NCHUNK_TOTAL // NW
K = 4
NGROUP = NCHUNK // K
BG = BATCH // C


@functools.partial(
    pl.kernel,
    mesh=plsc.VectorSubcoreMesh(core_axis_name="c", subcore_axis_name="s"),
    out_type=jax.ShapeDtypeStruct((HIST, 8, BG, 8, C), jnp.float32),
    scratch_types=[
        pltpu.VMEM((NCHUNK, C), jnp.int32),
        pltpu.VMEM((K, C, 128), jnp.float32),
        pltpu.VMEM((2, 8, 8, C), jnp.float32),
        pltpu.SemaphoreType.DMA,
        pltpu.SemaphoreType.DMA,
    ],
    compiler_params=pltpu.CompilerParams(
        use_tc_tiling_on_sc=False, needs_layout_passes=False
    ),
)
def _gather_rows(idx_hbm, table_hbm, out_hbm, idx_v, rows_v, tr_v, gsem, ssem):
    cid = lax.axis_index("c")
    sid = lax.axis_index("s")
    wid = sid * 2 + cid
    pltpu.sync_copy(idx_hbm.at[wid], idx_v)
    q0 = wid * NCHUNK

    lane = lax.iota(jnp.int32, 16)
    row_bases = [g * 16 + lane for g in range(8)]

    def transpose_chunk(b, t):
        rb = rows_v.at[b]

        def body(cg, carry):
            def load_burst(cs):
                col = cg * 8 + jnp.full((16,), cs, dtype=jnp.int32)
                return [
                    plsc.load_gather(rb, [row_bases[gg], col])
                    for gg in range(8)
                ]

            prev = load_burst(0)
            for cs in range(1, 8):
                cur = load_burst(cs)
                for gg in range(8):
                    tr_v[t, cg, cs - 1, pl.ds(gg * 16, 16)] = prev[gg]
                prev = cur
            for gg in range(8):
                tr_v[t, cg, 7, pl.ds(gg * 16, 16)] = prev[gg]
            return carry

        lax.fori_loop(0, 8, body, 0)

    def group(g, carry):
        j0 = g * K
        gathers = []
        for b in range(K):
            gathers.append(
                pltpu.async_copy(table_hbm.at[idx_v.at[j0 + b]], rows_v.at[b], gsem)
            )
        stores = []
        for b in range(K):
            t = b & 1
            gathers[b].wait()
            if b >= 2:
                stores[b - 2].wait()
            transpose_chunk(b, t)
            q = q0 + j0 + b
            h = q >> 7
            bg = q & 127
            stores.append(
                pltpu.async_copy(tr_v.at[t], out_hbm.at[h, :, bg], ssem)
            )
        for b in range(K - 2, K):
            stores[b].wait()
        return carry

    lax.fori_loop(0, NGROUP, group, 0)


def kernel(data, ivectors, ovectors):
    idx = data.astype(jnp.int32).T.reshape(NW, NCHUNK, C)
    tab = jnp.pad(ivectors, ((0, 0), (0, 128 - EMBED)))
    out5 = _gather_rows(idx, tab)
    return out5.transpose(2, 4, 0, 1, 3).reshape(BATCH, HIST, EMBED)


# speedup vs baseline: 1.0106x; 1.0106x over previous
import functools

import jax
import jax.numpy as jnp
from jax import lax
from jax.experimental import pallas as pl
from jax.experimental.pallas import tpu as pltpu
from jax.experimental.pallas import tpu_sc as plsc

VOCAB = 1000000
EMBED = 64
BATCH = 16384
HIST = 50

NW = 32
TOTAL = BATCH * HIST
C = 128
NCHUNK_TOTAL = TOTAL // C
NCHUNK = NCHUNK_TOTAL // NW
GB = 4
NGROUP = NCHUNK // GB
BG = BATCH // C


@functools.partial(
    pl.kernel,
    mesh=plsc.VectorSubcoreMesh(core_axis_name="c", subcore_axis_name="s"),
    out_type=jax.ShapeDtypeStruct((HIST, 8, BG, 8, C), jnp.float32),
    scratch_types=[
        pltpu.VMEM((NCHUNK, C), jnp.int32),
        pltpu.VMEM((GB, C, EMBED), jnp.float32),
        pltpu.VMEM((2, 8, GB, 8, C), jnp.float32),
        pltpu.SemaphoreType.DMA,
        pltpu.SemaphoreType.DMA,
        pltpu.SemaphoreType.DMA,
        pltpu.SemaphoreType.DMA,
        pltpu.SemaphoreType.DMA,
        pltpu.SemaphoreType.DMA,
    ],
    compiler_params=pltpu.CompilerParams(
        use_tc_tiling_on_sc=False, needs_layout_passes=False
    ),
)
def _gather_rows(idx_hbm, table_hbm, out_hbm,
                 idx_v, rows_v, tr_v, g0, g1, g2, g3, s0, s1):
    gsem = [g0, g1, g2, g3]
    ssem = [s0, s1]
    cid = lax.axis_index("c")
    sid = lax.axis_index("s")
    wid = sid * 2 + cid
    pltpu.sync_copy(idx_hbm.at[wid], idx_v)
    bg0 = wid * GB

    lane = lax.iota(jnp.int32, 16)
    row_bases = [g * 16 + lane for g in range(8)]

    def fire_gather(j, b):
        return pltpu.async_copy(table_hbm.at[idx_v.at[j]], rows_v.at[b], gsem[b])

    def wait_gather(b):
        pltpu.make_async_copy(
            table_hbm.at[idx_v.at[0]], rows_v.at[b], gsem[b]
        ).wait()

    def wait_store(p):
        pltpu.make_async_copy(
            tr_v.at[0, 0], out_hbm.at[0, 0, pl.ds(0, GB)], ssem[p]
        ).wait()

    def transpose_chunk(p, b):
        rb = rows_v.at[b]

        def body(cg, carry):
            def load_burst(cs):
                col = cg * 8 + jnp.full((16,), cs, dtype=jnp.int32)
                return [
                    plsc.load_gather(rb, [row_bases[gg], col])
                    for gg in range(8)
                ]

            prev = load_burst(0)
            for cs in range(1, 8):
                cur = load_burst(cs)
                for gg in range(8):
                    tr_v[p, cg, b, cs - 1, pl.ds(gg * 16, 16)] = prev[gg]
                prev = cur
            for gg in range(8):
                tr_v[p, cg, b, 7, pl.ds(gg * 16, 16)] = prev[gg]
            return carry

        lax.fori_loop(0, 8, body, 0)

    for b in range(GB):
        fire_gather(b, b)

    def pair(m, carry):
        for p in range(2):
            g = 2 * m + p
            @pl.when(m >= 1)
            def _():
                for _ in range(8):
                    wait_store(p)

            for b in range(GB):
                wait_gather(b)
                transpose_chunk(p, b)
                if p == 0:
                    fire_gather((g + 1) * GB + b, b)
                else:
                    @pl.when(m < NGROUP // 2 - 1)
                    def _():
                        fire_gather((g + 1) * GB + b, b)

            for cg in range(8):
                pltpu.async_copy(
                    tr_v.at[p, cg], out_hbm.at[g, cg, pl.ds(bg0, GB)], ssem[p]
                )
        return carry

    lax.fori_loop(0, NGROUP // 2, pair, 0)

    for p in range(2):
        for _ in range(8):
            wait_store(p)


def kernel(data, ivectors, ovectors):
    idx = (
        data.astype(jnp.int32).T
        .reshape(HIST, NW, GB, C)
        .transpose(1, 0, 2, 3)
        .reshape(NW, NCHUNK, C)
    )
    out5 = _gather_rows(idx, ivectors)
    return out5.transpose(2, 4, 0, 1, 3).reshape(BATCH, HIST, EMBED)
